# baseline (speedup 1.0000x reference)
import functools
import math

import jax
import jax.numpy as jnp
from jax.experimental import pallas as pl
from jax.experimental.pallas import tpu as pltpu

_INV_SQRT2 = 1.0 / math.sqrt(2.0)
_NSTRIPE = 4


def _router_kernel(sub_n, nbuf, x_hbm, w1_ref, b1_ref, w2_ref, b2_ref,
                   packets_ref, probs_ref, xbuf, dma_sems):
    i = pl.program_id(0)
    nsteps = pl.num_programs(0)

    def start_copy(step, slot):
        pltpu.make_async_copy(
            x_hbm.at[:, pl.ds(step * sub_n, sub_n), :],
            xbuf.at[slot],
            dma_sems.at[slot],
        ).start()

    @pl.when(i == 0)
    def _():
        for s in range(nbuf - 1):
            start_copy(s, s)

    refill = i + nbuf - 1

    @pl.when(refill < nsteps)
    def _():
        start_copy(refill, refill % nbuf)

    slot = jax.lax.rem(i, nbuf)
    pltpu.make_async_copy(
        x_hbm.at[:, pl.ds(i * sub_n, sub_n), :],
        xbuf.at[slot],
        dma_sems.at[slot],
    ).wait()

    w1 = w1_ref[...]
    b1 = b1_ref[...]
    w2 = w2_ref[...]
    b2 = b2_ref[...]
    for s in range(_NSTRIPE):
        h = jnp.dot(xbuf[slot, s], w1, preferred_element_type=jnp.float32)
        h = h + b1
        h = 0.5 * h * (1.0 + jax.lax.erf(h * _INV_SQRT2))
        logits = jnp.dot(h, w2, preferred_element_type=jnp.float32)
        logits = logits + b2
        packets_ref[s] = jnp.argmax(
            logits, axis=-1, keepdims=True).astype(jnp.int32)
        m = jnp.max(logits, axis=-1, keepdims=True)
        e = jnp.exp(logits - m)
        probs_ref[s] = e / jnp.sum(e, axis=-1, keepdims=True)


@functools.partial(jax.jit, static_argnames=("sub_n", "nbuf"))
def kernel(x, W1, b1, W2, b2, sub_n: int = 1024, nbuf: int = 3):
    n, d = x.shape
    h_dim = W1.shape[1]
    p = W2.shape[1]
    seg = n // _NSTRIPE
    nsteps = seg // sub_n
    x3 = x.reshape(_NSTRIPE, seg, d)
    packets3d, probs3d = pl.pallas_call(
        functools.partial(_router_kernel, sub_n, nbuf),
        grid=(nsteps,),
        in_specs=[
            pl.BlockSpec(memory_space=pltpu.MemorySpace.HBM),
            pl.BlockSpec((d, h_dim), lambda i: (0, 0)),
            pl.BlockSpec((h_dim,), lambda i: (0,)),
            pl.BlockSpec((h_dim, p), lambda i: (0, 0)),
            pl.BlockSpec((p,), lambda i: (0,)),
        ],
        out_specs=[
            pl.BlockSpec((_NSTRIPE, sub_n, 1), lambda i: (0, i, 0)),
            pl.BlockSpec((_NSTRIPE, sub_n, p), lambda i: (0, i, 0)),
        ],
        out_shape=[
            jax.ShapeDtypeStruct((_NSTRIPE, seg, 1), jnp.int32),
            jax.ShapeDtypeStruct((_NSTRIPE, seg, p), jnp.float32),
        ],
        scratch_shapes=[
            pltpu.VMEM((nbuf, _NSTRIPE, sub_n, d), jnp.float32),
            pltpu.SemaphoreType.DMA((nbuf,)),
        ],
        compiler_params=pltpu.CompilerParams(
            dimension_semantics=("arbitrary",),
        ),
    )(x3, W1, b1, W2, b2)
    return packets3d.reshape(n), probs3d.reshape(n, p)
